# Initial kernel scaffold; baseline (speedup 1.0000x reference)
import jax
import jax.numpy as jnp
from jax.experimental import pallas as pl


def kernel(g_edge_index, g_edge_rel, glob_edge_index, glob_edge_rel, rel_edge_index, seed_nodes, relation_batch, neighbor_batch_size, node_emb, global_emb, edge_emb):
    raise NotImplementedError("write your pallas kernel here")


# speedup vs baseline: 1.5252x; 1.5252x over previous
import functools

import jax
import jax.numpy as jnp
from jax import lax
from jax.experimental import pallas as pl
from jax.experimental.pallas import tpu as pltpu
from jax.experimental.pallas import tpu_sc as plsc

NC, NS, LN = 2, 16, 16
f32 = jnp.float32
i32 = jnp.int32


def _mesh():
    return plsc.VectorSubcoreMesh(
        core_axis_name="c", subcore_axis_name="s", num_cores=NC, num_subcores=NS
    )


def _rsqrt16(x):
    y = 1.0 / x
    for _ in range(28):
        y = y * (1.5 - 0.5 * x * y * y)
    return y


def _zero_1d(ref, n):
    def body(idx, _):
        ref[pl.ds(idx * LN, LN)] = jnp.zeros((LN,), f32)
        return 0

    lax.fori_loop(0, n // LN, body, 0)


def _make_deg_kernel(NR, NRT, ER):
    STRIPE = ER // NS
    NB = STRIPE // 128
    assert NB % 8 == 0 or NB == 8
    ZSPAN = NRT // NS
    ZB = ZSPAN if ZSPAN <= 6464 else ZSPAN // 2
    assert ZSPAN % ZB == 0 and ZB % LN == 0
    WSPAN = NR // NS
    WCH = 6400 if WSPAN % 6400 == 0 else WSPAN
    assert WSPAN % WCH == 0 and WCH % LN == 0

    @functools.partial(
        pl.kernel,
        out_type=jax.ShapeDtypeStruct((2, NR), f32),
        mesh=_mesh(),
        compiler_params=pltpu.CompilerParams(needs_layout_passes=False),
        scratch_types=[
            pltpu.VMEM_SHARED((NRT,), f32),
            pltpu.VMEM((STRIPE,), i32),
            [pltpu.VMEM((128,), i32) for _ in range(8)],
            pltpu.VMEM((128,), f32),
            pltpu.VMEM((ZB,), f32),
            pltpu.VMEM((WCH,), f32),
            pltpu.SemaphoreType.DMA,
        ],
    )
    def deg_kernel(edges_hbm, inv_hbm,
                   deg_sh, idx1_v, idxb_v, ones_v, zeros_v, val_v, sem):
        c = lax.axis_index("c")
        s = lax.axis_index("s")

        _zero_1d(zeros_v, ZB)

        def ob(idx, _):
            ones_v[pl.ds(idx * LN, LN)] = jnp.ones((LN,), f32)
            return 0

        lax.fori_loop(0, 128 // LN, ob, 0)

        for z in range(ZSPAN // ZB):
            pltpu.sync_copy(zeros_v, deg_sh.at[pl.ds(s * ZSPAN + z * ZB, ZB)])
        plsc.subcore_barrier()

        base = s * STRIPE
        pltpu.sync_copy(edges_hbm.at[c, pl.ds(base, STRIPE)], idx1_v)

        def grp(g, _):
            for k in range(8):
                b = g * 8 + k
                for v in range(128 // LN):
                    idxb_v[k][pl.ds(v * LN, LN)] = (
                        idx1_v[pl.ds(b * 128 + v * LN, LN)])
                pltpu.async_copy(ones_v, deg_sh.at[idxb_v[k]], sem, add=True)
            for k in range(8):
                pltpu.make_async_copy(ones_v, deg_sh.at[idxb_v[k]], sem).wait()
            return 0

        lax.fori_loop(0, NB // 8, grp, 0)
        plsc.subcore_barrier()

        def wo(w, _):
            off = s * WSPAN + w * WCH
            pltpu.sync_copy(deg_sh.at[pl.ds(off, WCH)], val_v)

            def rb(idx, _):
                d = val_v[pl.ds(idx * LN, LN)]
                val_v[pl.ds(idx * LN, LN)] = _rsqrt16(jnp.maximum(d, 1.0))
                return 0

            lax.fori_loop(0, WCH // LN, rb, 0)
            pltpu.sync_copy(val_v, inv_hbm.at[c, pl.ds(off, WCH)])
            return 0

        lax.fori_loop(0, WSPAN // WCH, wo, 0)

    return deg_kernel


def _make_gcn_kernel(Nh, NR, C, NCH, ER, use_rel, residual):
    STRIPE = ER // NS
    SEG = 1600 if STRIPE % 1600 == 0 else STRIPE
    NSEG = STRIPE // SEG
    BK = 64
    CB = SEG + BK
    EROWS = C // NS
    assert C * NCH == NR and EROWS % BK == 0 and SEG % LN == 0

    scratch = [
        pltpu.VMEM_SHARED((C + 8, 128), f32),
        pltpu.VMEM((STRIPE,), i32),
        pltpu.VMEM((CB,), i32),
        pltpu.VMEM((CB,), i32),
        pltpu.VMEM((BK,), i32),
        pltpu.VMEM((BK,), i32),
        pltpu.VMEM((BK, 128), f32),
        pltpu.VMEM((BK,), f32),
        pltpu.VMEM((16, 128), f32),
    ]
    if use_rel:
        scratch += [
            pltpu.VMEM_SHARED((1024, 128), f32),
            pltpu.VMEM((BK,), i32),
            pltpu.VMEM((BK, 128), f32),
        ]
    elif residual:
        scratch += [pltpu.VMEM((BK, 128), f32)]

    @functools.partial(
        pl.kernel,
        out_type=jax.ShapeDtypeStruct((NR, 128), f32),
        mesh=_mesh(),
        compiler_params=pltpu.CompilerParams(needs_layout_passes=False),
        scratch_types=scratch,
    )
    def gcn_kernel(*args):
        if use_rel:
            (h_hbm, src_hbm, dst_hbm, rel_hbm, invout_hbm, invin_hbm,
             relT_hbm, out_hbm,
             agg_sh, dstrip, cld, cpos, idxb, srcb, hrows,
             scal_v, zeros_v, relT_sh, relb, rrows) = args
        elif residual:
            (h_hbm, src_hbm, dst_hbm, invout_hbm, invin_hbm, out_hbm,
             agg_sh, dstrip, cld, cpos, idxb, srcb, hrows,
             scal_v, zeros_v, rrows) = args
        else:
            (h_hbm, src_hbm, dst_hbm, invout_hbm, invin_hbm, out_hbm,
             agg_sh, dstrip, cld, cpos, idxb, srcb, hrows,
             scal_v, zeros_v) = args

        c = lax.axis_index("c")
        s = lax.axis_index("s")

        def zb(r, _):
            for j in range(8):
                zeros_v[r, pl.ds(j * LN, LN)] = jnp.zeros((LN,), f32)
            return 0

        lax.fori_loop(0, 16, zb, 0)

        if use_rel:
            pltpu.sync_copy(relT_hbm.at[pl.ds(s * 64, 64)],
                            relT_sh.at[pl.ds(s * 64, 64)])
        base = s * STRIPE
        pltpu.sync_copy(dst_hbm.at[pl.ds(base, STRIPE)], dstrip)
        plsc.subcore_barrier()

        def chunk(g, _):
            lo = g * C
            active = lax.rem(g, 2) == c

            @pl.when(active)
            def _():
                def azb(k, _):
                    pltpu.sync_copy(
                        zeros_v, agg_sh.at[pl.ds(s * EROWS + k * 16, 16)])
                    return 0

                lax.fori_loop(0, EROWS // 16, azb, 0)

            plsc.subcore_barrier()

            @pl.when(active)
            def _():
                def seg(t, _):
                    def cb(v, cnt):
                        off = t * SEG + v * LN
                        d = dstrip[pl.ds(off, LN)]
                        m = (d >= lo) & (d < lo + C)
                        plsc.store_compressed(cld.at[pl.ds(cnt, LN)], d - lo,
                                              mask=m)
                        pos = jnp.full((LN,), base + off, i32) + lax.iota(i32, LN)
                        plsc.store_compressed(cpos.at[pl.ds(cnt, LN)], pos,
                                              mask=m)
                        return cnt + jnp.sum(jnp.where(m, 1, 0))

                    cnt = lax.fori_loop(0, SEG // LN, cb, 0)
                    for k in range(BK // LN):
                        cld[pl.ds(cnt + k * LN, LN)] = jnp.full((LN,), C, i32)
                        cpos[pl.ds(cnt + k * LN, LN)] = jnp.full(
                            (LN,), base, i32)
                    kb = (cnt + BK - 1) // BK

                    def blk(k, _):
                        boff = k * BK
                        pltpu.sync_copy(src_hbm.at[cpos.at[pl.ds(boff, BK)]],
                                        srcb)
                        pltpu.sync_copy(h_hbm.at[srcb], hrows)
                        pltpu.sync_copy(invout_hbm.at[srcb], scal_v)
                        if use_rel:
                            pltpu.sync_copy(
                                rel_hbm.at[cpos.at[pl.ds(boff, BK)]], relb)
                            pltpu.sync_copy(relT_sh.at[relb], rrows)

                        def rowm(gr, _):
                            scv = scal_v[pl.ds(gr * LN, LN)]
                            for r16 in range(LN):
                                r = gr * LN + r16
                                sc = jnp.broadcast_to(scv[r16], (LN,))
                                for j in range(8):
                                    x = hrows[r, pl.ds(j * LN, LN)] * sc
                                    if use_rel:
                                        x = x * rrows[r, pl.ds(j * LN, LN)]
                                    hrows[r, pl.ds(j * LN, LN)] = x
                            return 0

                        lax.fori_loop(0, BK // LN, rowm, 0)
                        for v in range(BK // LN):
                            idxb[pl.ds(v * LN, LN)] = (
                                cld[pl.ds(boff + v * LN, LN)])
                        pltpu.sync_copy(hrows, agg_sh.at[idxb], add=True)
                        return 0

                    lax.fori_loop(0, kb, blk, 0)
                    return 0

                lax.fori_loop(0, NSEG, seg, 0)

            plsc.subcore_barrier()

            @pl.when(active)
            def _():
                def eb(k, _):
                    row0 = lo + s * EROWS + k * BK
                    pltpu.sync_copy(agg_sh.at[pl.ds(s * EROWS + k * BK, BK)],
                                    hrows)
                    pltpu.sync_copy(invin_hbm.at[pl.ds(row0, BK)], scal_v)
                    if residual:
                        hadr = jnp.minimum(row0, Nh - BK)
                        pltpu.sync_copy(h_hbm.at[pl.ds(hadr, BK)], rrows)

                    def rowm(gr, _):
                        scv = scal_v[pl.ds(gr * LN, LN)]
                        for r16 in range(LN):
                            r = gr * LN + r16
                            iv = jnp.broadcast_to(scv[r16], (LN,))
                            for j in range(8):
                                x = hrows[r, pl.ds(j * LN, LN)] * iv
                                if residual:
                                    x = x + rrows[r, pl.ds(j * LN, LN)]
                                hrows[r, pl.ds(j * LN, LN)] = x
                        return 0

                    lax.fori_loop(0, BK // LN, rowm, 0)
                    pltpu.sync_copy(hrows, out_hbm.at[pl.ds(row0, BK)])
                    return 0

                lax.fori_loop(0, EROWS // BK, eb, 0)

            return 0

        lax.fori_loop(0, NCH, chunk, 0)

    return gcn_kernel


def _make_final_kernel():
    B = 4096
    PW = B // (NC * NS)

    @functools.partial(
        pl.kernel,
        out_type=(
            jax.ShapeDtypeStruct((B, 128), f32),
            jax.ShapeDtypeStruct((B, 128), f32),
        ),
        mesh=_mesh(),
        compiler_params=pltpu.CompilerParams(needs_layout_passes=False),
        scratch_types=[
            pltpu.VMEM((PW,), i32),
            pltpu.VMEM((PW,), i32),
            pltpu.VMEM((PW, 128), f32),
            pltpu.VMEM((PW, 128), f32),
        ],
    )
    def final_kernel(ht_hbm, hg_hbm, er_hbm, ee_hbm, seed_hbm, rb_hbm,
                     n_hbm, e_hbm, ia_v, ib_v, a_v, b_v):
        c = lax.axis_index("c")
        s = lax.axis_index("s")
        wid = s * NC + c
        base = wid * PW

        def mix(out_hbm):
            def rowm(r, _):
                for j in range(8):
                    a_v[r, pl.ds(j * LN, LN)] = (
                        a_v[r, pl.ds(j * LN, LN)] * 0.5
                        + b_v[r, pl.ds(j * LN, LN)] * 0.5)
                return 0

            lax.fori_loop(0, PW, rowm, 0)
            pltpu.sync_copy(a_v, out_hbm.at[pl.ds(base, PW)])

        pltpu.sync_copy(seed_hbm.at[pl.ds(base, PW)], ia_v)

        def ob(idx, _):
            sv = ia_v[pl.ds(idx * LN, LN)]
            ib_v[pl.ds(idx * LN, LN)] = lax.shift_right_logical(sv, 2)
            return 0

        lax.fori_loop(0, PW // LN, ob, 0)
        pltpu.sync_copy(ht_hbm.at[ia_v], a_v)
        pltpu.sync_copy(hg_hbm.at[ib_v], b_v)
        mix(n_hbm)

        pltpu.sync_copy(rb_hbm.at[pl.ds(base, PW)], ia_v)
        pltpu.sync_copy(er_hbm.at[ia_v], a_v)
        pltpu.sync_copy(ee_hbm.at[ia_v], b_v)
        mix(e_hbm)

    return final_kernel


_NT, _NG, _NRL = 200000, 50000, 1000
_C = 8192
_NR_T, _NCH_T = 204800, 25
_NR_G, _NCH_G = 57344, 7
_NR_R, _NCH_R = 1024, 1
_ER_BIG, _ER_REL = 409600, 16384

_deg_t = _make_deg_kernel(_NR_T, _NR_T + 2048, _ER_BIG)
_deg_g = _make_deg_kernel(_NR_G, _NR_G + 2048, _ER_BIG)
_deg_r = _make_deg_kernel(_NR_R, _NR_R + 2048, _ER_REL)
_gcn_t = _make_gcn_kernel(_NT, _NR_T, _C, _NCH_T, _ER_BIG, True, True)
_gcn_g = _make_gcn_kernel(_NG, _NR_G, _C, _NCH_G, _ER_BIG, True, True)
_gcn_r = _make_gcn_kernel(_NR_R, _NR_R, _NR_R, _NCH_R, _ER_REL, False, False)
_final = _make_final_kernel()


def _pad_edges(src, dst, rel, er, dummy):
    pe = er - src.shape[0]
    pad_i = jnp.full((pe,), dummy, i32)
    out = (jnp.concatenate([src, pad_i]), jnp.concatenate([dst, pad_i]))
    if rel is not None:
        out += (jnp.concatenate([rel, jnp.zeros((pe,), i32)]),)
    return out


def kernel(g_edge_index, g_edge_rel, glob_edge_index, glob_edge_rel,
           rel_edge_index, seed_nodes, relation_batch, neighbor_batch_size,
           node_emb, global_emb, edge_emb):
    del neighbor_batch_size
    tsrc, tdst, trel = _pad_edges(
        g_edge_index[0], g_edge_index[1], g_edge_rel, _ER_BIG, _NR_T)
    gsrc, gdst, grel = _pad_edges(
        glob_edge_index[0], glob_edge_index[1], glob_edge_rel, _ER_BIG, _NR_G)
    rsrc, rdst = _pad_edges(
        rel_edge_index[0], rel_edge_index[1], None, _ER_REL, _NRL)
    ee_pad = jnp.concatenate(
        [edge_emb, jnp.zeros((_NR_R - _NRL, 128), f32)], axis=0)

    t_inv = _deg_t(jnp.stack([tsrc, tdst]))
    g_inv = _deg_g(jnp.stack([gsrc, gdst]))
    r_inv = _deg_r(jnp.stack([rsrc, rdst]))
    t_io, t_ii = t_inv[0], t_inv[1]
    g_io, g_ii = g_inv[0], g_inv[1]
    r_io, r_ii = r_inv[0], r_inv[1]

    ht1 = _gcn_t(node_emb, tsrc, tdst, trel, t_io, t_ii, ee_pad)
    ht2 = _gcn_t(ht1, tsrc, tdst, trel, t_io, t_ii, ee_pad)
    hg1 = _gcn_g(global_emb, gsrc, gdst, grel, g_io, g_ii, ee_pad)
    hg2 = _gcn_g(hg1, gsrc, gdst, grel, g_io, g_ii, ee_pad)
    er = _gcn_r(ee_pad, rsrc, rdst, r_io, r_ii)

    n, e = _final(ht2, hg2, er, ee_pad, seed_nodes, relation_batch)
    return n, e


# speedup vs baseline: 1.7880x; 1.1723x over previous
import functools

import jax
import jax.numpy as jnp
from jax import lax
from jax.experimental import pallas as pl
from jax.experimental.pallas import tpu as pltpu
from jax.experimental.pallas import tpu_sc as plsc

NC, NS, LN = 2, 16, 16
f32 = jnp.float32
i32 = jnp.int32


def _mesh():
    return plsc.VectorSubcoreMesh(
        core_axis_name="c", subcore_axis_name="s", num_cores=NC, num_subcores=NS
    )


def _rsqrt16(x):
    y = 1.0 / x
    for _ in range(28):
        y = y * (1.5 - 0.5 * x * y * y)
    return y


def _zero_1d(ref, n):
    def body(idx, _):
        ref[pl.ds(idx * LN, LN)] = jnp.zeros((LN,), f32)
        return 0

    lax.fori_loop(0, n // LN, body, 0)


def _make_deg_kernel(NR, NRT, ER):
    STRIPE = ER // NS
    NB = STRIPE // 128
    assert NB % 8 == 0 or NB == 8
    ZSPAN = NRT // NS
    ZB = ZSPAN if ZSPAN <= 6464 else ZSPAN // 2
    assert ZSPAN % ZB == 0 and ZB % LN == 0
    WSPAN = NR // NS
    WCH = 6400 if WSPAN % 6400 == 0 else WSPAN
    assert WSPAN % WCH == 0 and WCH % LN == 0

    @functools.partial(
        pl.kernel,
        out_type=jax.ShapeDtypeStruct((2, NR), f32),
        mesh=_mesh(),
        compiler_params=pltpu.CompilerParams(needs_layout_passes=False),
        scratch_types=[
            pltpu.VMEM_SHARED((NRT,), f32),
            pltpu.VMEM((STRIPE,), i32),
            [pltpu.VMEM((128,), i32) for _ in range(8)],
            pltpu.VMEM((128,), f32),
            pltpu.VMEM((ZB,), f32),
            pltpu.VMEM((WCH,), f32),
            pltpu.SemaphoreType.DMA,
        ],
    )
    def deg_kernel(edges_hbm, inv_hbm,
                   deg_sh, idx1_v, idxb_v, ones_v, zeros_v, val_v, sem):
        c = lax.axis_index("c")
        s = lax.axis_index("s")

        _zero_1d(zeros_v, ZB)

        def ob(idx, _):
            ones_v[pl.ds(idx * LN, LN)] = jnp.ones((LN,), f32)
            return 0

        lax.fori_loop(0, 128 // LN, ob, 0)

        for z in range(ZSPAN // ZB):
            pltpu.sync_copy(zeros_v, deg_sh.at[pl.ds(s * ZSPAN + z * ZB, ZB)])
        plsc.subcore_barrier()

        base = s * STRIPE
        pltpu.sync_copy(edges_hbm.at[c, pl.ds(base, STRIPE)], idx1_v)

        def grp(g, _):
            for k in range(8):
                b = g * 8 + k
                for v in range(128 // LN):
                    idxb_v[k][pl.ds(v * LN, LN)] = (
                        idx1_v[pl.ds(b * 128 + v * LN, LN)])
                pltpu.async_copy(ones_v, deg_sh.at[idxb_v[k]], sem, add=True)
            for k in range(8):
                pltpu.make_async_copy(ones_v, deg_sh.at[idxb_v[k]], sem).wait()
            return 0

        lax.fori_loop(0, NB // 8, grp, 0)
        plsc.subcore_barrier()

        def wo(w, _):
            off = s * WSPAN + w * WCH
            pltpu.sync_copy(deg_sh.at[pl.ds(off, WCH)], val_v)

            def rb(idx, _):
                d = val_v[pl.ds(idx * LN, LN)]
                val_v[pl.ds(idx * LN, LN)] = _rsqrt16(jnp.maximum(d, 1.0))
                return 0

            lax.fori_loop(0, WCH // LN, rb, 0)
            pltpu.sync_copy(val_v, inv_hbm.at[c, pl.ds(off, WCH)])
            return 0

        lax.fori_loop(0, WSPAN // WCH, wo, 0)

    return deg_kernel


def _make_gcn_kernel(Nh, NR, C, NCH, ER, use_rel, residual):
    STRIPE = ER // NS
    SEG = 3200 if STRIPE % 3200 == 0 else STRIPE
    NSEG = STRIPE // SEG
    BK = 64
    CB = SEG + BK
    EROWS = C // NS
    assert C * NCH == NR and EROWS % BK == 0 and SEG % LN == 0

    scratch = [
        pltpu.VMEM_SHARED((C + 8, 128), f32),
        pltpu.VMEM((STRIPE,), i32),
        pltpu.VMEM((CB,), i32),
        pltpu.VMEM((CB,), i32),
        pltpu.VMEM((BK,), i32),
        pltpu.VMEM((BK,), i32),
        pltpu.VMEM((BK, 128), f32),
        pltpu.VMEM((BK,), f32),
        pltpu.VMEM((16, 128), f32),
    ]
    if use_rel:
        scratch += [
            pltpu.VMEM_SHARED((1024, 128), f32),
            pltpu.VMEM((BK,), i32),
            pltpu.VMEM((BK, 128), f32),
        ]
    elif residual:
        scratch += [pltpu.VMEM((BK, 128), f32)]

    @functools.partial(
        pl.kernel,
        out_type=jax.ShapeDtypeStruct((NR, 128), f32),
        mesh=_mesh(),
        compiler_params=pltpu.CompilerParams(needs_layout_passes=False),
        scratch_types=scratch,
    )
    def gcn_kernel(*args):
        if use_rel:
            (h_hbm, src_hbm, dst_hbm, rel_hbm, invout_hbm, invin_hbm,
             relT_hbm, out_hbm,
             agg_sh, dstrip, cld, cpos, idxb, srcb, hrows,
             scal_v, zeros_v, relT_sh, relb, rrows) = args
        elif residual:
            (h_hbm, src_hbm, dst_hbm, invout_hbm, invin_hbm, out_hbm,
             agg_sh, dstrip, cld, cpos, idxb, srcb, hrows,
             scal_v, zeros_v, rrows) = args
        else:
            (h_hbm, src_hbm, dst_hbm, invout_hbm, invin_hbm, out_hbm,
             agg_sh, dstrip, cld, cpos, idxb, srcb, hrows,
             scal_v, zeros_v) = args

        c = lax.axis_index("c")
        s = lax.axis_index("s")

        def zb(r, _):
            for j in range(8):
                zeros_v[r, pl.ds(j * LN, LN)] = jnp.zeros((LN,), f32)
            return 0

        lax.fori_loop(0, 16, zb, 0)

        if use_rel:
            pltpu.sync_copy(relT_hbm.at[pl.ds(s * 64, 64)],
                            relT_sh.at[pl.ds(s * 64, 64)])
        base = s * STRIPE
        pltpu.sync_copy(dst_hbm.at[pl.ds(base, STRIPE)], dstrip)
        plsc.subcore_barrier()

        def chunk(g, _):
            lo = g * C
            active = lax.rem(g, 2) == c

            @pl.when(active)
            def _():
                def azb(k, _):
                    pltpu.sync_copy(
                        zeros_v, agg_sh.at[pl.ds(s * EROWS + k * 16, 16)])
                    return 0

                lax.fori_loop(0, EROWS // 16, azb, 0)

            plsc.subcore_barrier()

            @pl.when(active)
            def _():
                def seg(t, _):
                    def cb(v, cnt):
                        off = t * SEG + v * LN
                        d = dstrip[pl.ds(off, LN)]
                        m = (d >= lo) & (d < lo + C)
                        plsc.store_compressed(cld.at[pl.ds(cnt, LN)], d - lo,
                                              mask=m)
                        pos = jnp.full((LN,), base + off, i32) + lax.iota(i32, LN)
                        plsc.store_compressed(cpos.at[pl.ds(cnt, LN)], pos,
                                              mask=m)
                        return cnt + jnp.sum(jnp.where(m, 1, 0))

                    cnt = lax.fori_loop(0, SEG // LN, cb, 0)
                    for k in range(BK // LN):
                        cld[pl.ds(cnt + k * LN, LN)] = jnp.full((LN,), C, i32)
                        cpos[pl.ds(cnt + k * LN, LN)] = jnp.full(
                            (LN,), base, i32)
                    kb = (cnt + BK - 1) // BK

                    def blk(k, _):
                        boff = k * BK
                        pltpu.sync_copy(src_hbm.at[cpos.at[pl.ds(boff, BK)]],
                                        srcb)
                        pltpu.sync_copy(h_hbm.at[srcb], hrows)
                        pltpu.sync_copy(invout_hbm.at[srcb], scal_v)
                        if use_rel:
                            pltpu.sync_copy(
                                rel_hbm.at[cpos.at[pl.ds(boff, BK)]], relb)
                            pltpu.sync_copy(relT_sh.at[relb], rrows)

                        def rowm(gr, _):
                            scv = scal_v[pl.ds(gr * LN, LN)]
                            for r16 in range(LN):
                                r = gr * LN + r16
                                sc = jnp.broadcast_to(scv[r16], (LN,))
                                for j in range(8):
                                    x = hrows[r, pl.ds(j * LN, LN)] * sc
                                    if use_rel:
                                        x = x * rrows[r, pl.ds(j * LN, LN)]
                                    hrows[r, pl.ds(j * LN, LN)] = x
                            return 0

                        lax.fori_loop(0, BK // LN, rowm, 0)
                        for v in range(BK // LN):
                            idxb[pl.ds(v * LN, LN)] = (
                                cld[pl.ds(boff + v * LN, LN)])
                        pltpu.sync_copy(hrows, agg_sh.at[idxb], add=True)
                        return 0

                    lax.fori_loop(0, kb, blk, 0)
                    return 0

                lax.fori_loop(0, NSEG, seg, 0)

            plsc.subcore_barrier()

            @pl.when(active)
            def _():
                def eb(k, _):
                    row0 = lo + s * EROWS + k * BK
                    pltpu.sync_copy(agg_sh.at[pl.ds(s * EROWS + k * BK, BK)],
                                    hrows)
                    pltpu.sync_copy(invin_hbm.at[pl.ds(row0, BK)], scal_v)
                    if residual:
                        for q in range(BK // LN):
                            hq = jnp.minimum(row0 + q * LN, Nh - LN)
                            pltpu.sync_copy(h_hbm.at[pl.ds(hq, LN)],
                                            rrows.at[pl.ds(q * LN, LN)])

                    def rowm(gr, _):
                        scv = scal_v[pl.ds(gr * LN, LN)]
                        for r16 in range(LN):
                            r = gr * LN + r16
                            iv = jnp.broadcast_to(scv[r16], (LN,))
                            for j in range(8):
                                x = hrows[r, pl.ds(j * LN, LN)] * iv
                                if residual:
                                    x = x + rrows[r, pl.ds(j * LN, LN)]
                                hrows[r, pl.ds(j * LN, LN)] = x
                        return 0

                    lax.fori_loop(0, BK // LN, rowm, 0)
                    pltpu.sync_copy(hrows, out_hbm.at[pl.ds(row0, BK)])
                    return 0

                lax.fori_loop(0, EROWS // BK, eb, 0)

            return 0

        lax.fori_loop(0, NCH, chunk, 0)

    return gcn_kernel


def _make_final_kernel():
    B = 4096
    PW = B // (NC * NS)

    @functools.partial(
        pl.kernel,
        out_type=(
            jax.ShapeDtypeStruct((B, 128), f32),
            jax.ShapeDtypeStruct((B, 128), f32),
        ),
        mesh=_mesh(),
        compiler_params=pltpu.CompilerParams(needs_layout_passes=False),
        scratch_types=[
            pltpu.VMEM((PW,), i32),
            pltpu.VMEM((PW,), i32),
            pltpu.VMEM((PW, 128), f32),
            pltpu.VMEM((PW, 128), f32),
        ],
    )
    def final_kernel(ht_hbm, hg_hbm, er_hbm, ee_hbm, seed_hbm, rb_hbm,
                     n_hbm, e_hbm, ia_v, ib_v, a_v, b_v):
        c = lax.axis_index("c")
        s = lax.axis_index("s")
        wid = s * NC + c
        base = wid * PW

        def mix(out_hbm):
            def rowm(r, _):
                for j in range(8):
                    a_v[r, pl.ds(j * LN, LN)] = (
                        a_v[r, pl.ds(j * LN, LN)] * 0.5
                        + b_v[r, pl.ds(j * LN, LN)] * 0.5)
                return 0

            lax.fori_loop(0, PW, rowm, 0)
            pltpu.sync_copy(a_v, out_hbm.at[pl.ds(base, PW)])

        pltpu.sync_copy(seed_hbm.at[pl.ds(base, PW)], ia_v)

        def ob(idx, _):
            sv = ia_v[pl.ds(idx * LN, LN)]
            ib_v[pl.ds(idx * LN, LN)] = lax.shift_right_logical(sv, 2)
            return 0

        lax.fori_loop(0, PW // LN, ob, 0)
        pltpu.sync_copy(ht_hbm.at[ia_v], a_v)
        pltpu.sync_copy(hg_hbm.at[ib_v], b_v)
        mix(n_hbm)

        pltpu.sync_copy(rb_hbm.at[pl.ds(base, PW)], ia_v)
        pltpu.sync_copy(er_hbm.at[ia_v], a_v)
        pltpu.sync_copy(ee_hbm.at[ia_v], b_v)
        mix(e_hbm)

    return final_kernel


_NT, _NG, _NRL = 200000, 50000, 1000
_C = 8192
_NR_T, _NCH_T = 204800, 25
_NR_G, _NCH_G = 57344, 7
_NR_R, _NCH_R = 1024, 1
_ER_BIG, _ER_REL = 409600, 16384

_deg_t = _make_deg_kernel(_NR_T, _NR_T + 2048, _ER_BIG)
_deg_g = _make_deg_kernel(_NR_G, _NR_G + 2048, _ER_BIG)
_deg_r = _make_deg_kernel(_NR_R, _NR_R + 2048, _ER_REL)
_gcn_t = _make_gcn_kernel(_NT, _NR_T, _C, _NCH_T, _ER_BIG, True, True)
_gcn_g = _make_gcn_kernel(_NG, _NR_G, _C, _NCH_G, _ER_BIG, True, True)
_gcn_r = _make_gcn_kernel(_NR_R, _NR_R, _NR_R, _NCH_R, _ER_REL, False, False)
_final = _make_final_kernel()


def _pad_edges(src, dst, rel, er, dummy):
    pe = er - src.shape[0]
    pad_i = jnp.full((pe,), dummy, i32)
    out = (jnp.concatenate([src, pad_i]), jnp.concatenate([dst, pad_i]))
    if rel is not None:
        out += (jnp.concatenate([rel, jnp.zeros((pe,), i32)]),)
    return out


def kernel(g_edge_index, g_edge_rel, glob_edge_index, glob_edge_rel,
           rel_edge_index, seed_nodes, relation_batch, neighbor_batch_size,
           node_emb, global_emb, edge_emb):
    del neighbor_batch_size
    tsrc, tdst, trel = _pad_edges(
        g_edge_index[0], g_edge_index[1], g_edge_rel, _ER_BIG, _NR_T)
    gsrc, gdst, grel = _pad_edges(
        glob_edge_index[0], glob_edge_index[1], glob_edge_rel, _ER_BIG, _NR_G)
    rsrc, rdst = _pad_edges(
        rel_edge_index[0], rel_edge_index[1], None, _ER_REL, _NRL)
    ee_pad = jnp.concatenate(
        [edge_emb, jnp.zeros((_NR_R - _NRL, 128), f32)], axis=0)

    t_inv = _deg_t(jnp.stack([tsrc, tdst]))
    g_inv = _deg_g(jnp.stack([gsrc, gdst]))
    r_inv = _deg_r(jnp.stack([rsrc, rdst]))
    t_io, t_ii = t_inv[0], t_inv[1]
    g_io, g_ii = g_inv[0], g_inv[1]
    r_io, r_ii = r_inv[0], r_inv[1]

    ht1 = _gcn_t(node_emb, tsrc, tdst, trel, t_io, t_ii, ee_pad)
    ht2 = _gcn_t(ht1, tsrc, tdst, trel, t_io, t_ii, ee_pad)
    hg1 = _gcn_g(global_emb, gsrc, gdst, grel, g_io, g_ii, ee_pad)
    hg2 = _gcn_g(hg1, gsrc, gdst, grel, g_io, g_ii, ee_pad)
    er = _gcn_r(ee_pad, rsrc, rdst, r_io, r_ii)

    n, e = _final(ht2, hg2, er, ee_pad, seed_nodes, relation_batch)
    return n, e


# speedup vs baseline: 2.7268x; 1.5251x over previous
import functools

import jax
import jax.numpy as jnp
from jax import lax
from jax.experimental import pallas as pl
from jax.experimental.pallas import tpu as pltpu
from jax.experimental.pallas import tpu_sc as plsc

NC, NS, LN = 2, 16, 16
f32 = jnp.float32
i32 = jnp.int32


def _mesh():
    return plsc.VectorSubcoreMesh(
        core_axis_name="c", subcore_axis_name="s", num_cores=NC, num_subcores=NS
    )


def _rsqrt16(x):
    y = 1.0 / x
    for _ in range(28):
        y = y * (1.5 - 0.5 * x * y * y)
    return y


def _zero_1d(ref, n):
    def body(idx, _):
        ref[pl.ds(idx * LN, LN)] = jnp.zeros((LN,), f32)
        return 0

    lax.fori_loop(0, n // LN, body, 0)


def _make_deg_kernel(NR, NRT, ER):
    STRIPE = ER // NS
    NB = STRIPE // 128
    assert NB % 8 == 0 or NB == 8
    ZSPAN = NRT // NS
    ZB = ZSPAN if ZSPAN <= 6464 else ZSPAN // 2
    assert ZSPAN % ZB == 0 and ZB % LN == 0
    WSPAN = NR // NS
    WCH = 6400 if WSPAN % 6400 == 0 else WSPAN
    assert WSPAN % WCH == 0 and WCH % LN == 0

    @functools.partial(
        pl.kernel,
        out_type=jax.ShapeDtypeStruct((2, NR), f32),
        mesh=_mesh(),
        compiler_params=pltpu.CompilerParams(needs_layout_passes=False),
        scratch_types=[
            pltpu.VMEM_SHARED((NRT,), f32),
            pltpu.VMEM((STRIPE,), i32),
            [pltpu.VMEM((128,), i32) for _ in range(8)],
            pltpu.VMEM((128,), f32),
            pltpu.VMEM((ZB,), f32),
            pltpu.VMEM((WCH,), f32),
            pltpu.SemaphoreType.DMA,
        ],
    )
    def deg_kernel(edges_hbm, inv_hbm,
                   deg_sh, idx1_v, idxb_v, ones_v, zeros_v, val_v, sem):
        c = lax.axis_index("c")
        s = lax.axis_index("s")

        _zero_1d(zeros_v, ZB)

        def ob(idx, _):
            ones_v[pl.ds(idx * LN, LN)] = jnp.ones((LN,), f32)
            return 0

        lax.fori_loop(0, 128 // LN, ob, 0)

        for z in range(ZSPAN // ZB):
            pltpu.sync_copy(zeros_v, deg_sh.at[pl.ds(s * ZSPAN + z * ZB, ZB)])
        plsc.subcore_barrier()

        base = s * STRIPE
        pltpu.sync_copy(edges_hbm.at[c, pl.ds(base, STRIPE)], idx1_v)

        def grp(g, _):
            for k in range(8):
                b = g * 8 + k
                for v in range(128 // LN):
                    idxb_v[k][pl.ds(v * LN, LN)] = (
                        idx1_v[pl.ds(b * 128 + v * LN, LN)])
                pltpu.async_copy(ones_v, deg_sh.at[idxb_v[k]], sem, add=True)
            for k in range(8):
                pltpu.make_async_copy(ones_v, deg_sh.at[idxb_v[k]], sem).wait()
            return 0

        lax.fori_loop(0, NB // 8, grp, 0)
        plsc.subcore_barrier()

        def wo(w, _):
            off = s * WSPAN + w * WCH
            pltpu.sync_copy(deg_sh.at[pl.ds(off, WCH)], val_v)

            def rb(idx, _):
                d = val_v[pl.ds(idx * LN, LN)]
                val_v[pl.ds(idx * LN, LN)] = _rsqrt16(jnp.maximum(d, 1.0))
                return 0

            lax.fori_loop(0, WCH // LN, rb, 0)
            pltpu.sync_copy(val_v, inv_hbm.at[c, pl.ds(off, WCH)])
            return 0

        lax.fori_loop(0, WSPAN // WCH, wo, 0)

    return deg_kernel


def _make_gcn_kernel(Nh, NR, C, NCH, ER, use_rel, residual):
    STRIPE = ER // NS
    SEG = 1600 if STRIPE % 1600 == 0 else STRIPE
    NSEG = STRIPE // SEG
    BK = 64
    CB = SEG + BK
    EROWS = C // NS
    assert C * NCH == NR and EROWS % BK == 0 and SEG % LN == 0

    scratch = [
        pltpu.VMEM_SHARED((C + 8, 128), f32),
        pltpu.VMEM((STRIPE,), i32),
        pltpu.VMEM((CB,), i32),
        pltpu.VMEM((CB,), i32),
        pltpu.VMEM((BK,), i32),
        pltpu.VMEM((BK,), i32),
        pltpu.VMEM((BK, 128), f32),
        pltpu.VMEM((BK,), f32),
        pltpu.VMEM((16, 128), f32),
    ]
    if use_rel:
        scratch += [
            pltpu.VMEM_SHARED((1024, 128), f32),
            pltpu.VMEM((BK,), i32),
            pltpu.VMEM((BK, 128), f32),
        ]
    elif residual:
        scratch += [pltpu.VMEM((BK, 128), f32)]

    @functools.partial(
        pl.kernel,
        out_type=jax.ShapeDtypeStruct((NR, 128), f32),
        mesh=_mesh(),
        compiler_params=pltpu.CompilerParams(needs_layout_passes=False),
        scratch_types=scratch,
    )
    def gcn_kernel(*args):
        if use_rel:
            (h_hbm, src_hbm, dst_hbm, rel_hbm, invout_hbm, invin_hbm,
             relT_hbm, out_hbm,
             agg_sh, dstrip, cld, cpos, idxb, srcb, hrows,
             scal_v, zeros_v, relT_sh, relb, rrows) = args
        elif residual:
            (h_hbm, src_hbm, dst_hbm, invout_hbm, invin_hbm, out_hbm,
             agg_sh, dstrip, cld, cpos, idxb, srcb, hrows,
             scal_v, zeros_v, rrows) = args
        else:
            (h_hbm, src_hbm, dst_hbm, invout_hbm, invin_hbm, out_hbm,
             agg_sh, dstrip, cld, cpos, idxb, srcb, hrows,
             scal_v, zeros_v) = args

        c = lax.axis_index("c")
        s = lax.axis_index("s")

        def zb(r, _):
            for j in range(8):
                zeros_v[r, pl.ds(j * LN, LN)] = jnp.zeros((LN,), f32)
            return 0

        lax.fori_loop(0, 16, zb, 0)

        if use_rel:
            pltpu.sync_copy(relT_hbm.at[pl.ds(s * 64, 64)],
                            relT_sh.at[pl.ds(s * 64, 64)])
        base = s * STRIPE
        pltpu.sync_copy(dst_hbm.at[pl.ds(base, STRIPE)], dstrip)
        plsc.subcore_barrier()

        def chunk(g, _):
            lo = g * C
            active = lax.rem(g, 2) == c

            @pl.when(active)
            def _():
                def azb(k, _):
                    pltpu.sync_copy(
                        zeros_v, agg_sh.at[pl.ds(s * EROWS + k * 16, 16)])
                    return 0

                lax.fori_loop(0, EROWS // 16, azb, 0)

            plsc.subcore_barrier()

            @pl.when(active)
            def _():
                def seg(t, _):
                    def cb(v, cnt):
                        off = t * SEG + v * LN
                        d = dstrip[pl.ds(off, LN)]
                        m = (d >= lo) & (d < lo + C)
                        plsc.store_compressed(cld.at[pl.ds(cnt, LN)], d - lo,
                                              mask=m)
                        pos = jnp.full((LN,), base + off, i32) + lax.iota(i32, LN)
                        plsc.store_compressed(cpos.at[pl.ds(cnt, LN)], pos,
                                              mask=m)
                        return cnt + jnp.sum(jnp.where(m, 1, 0))

                    cnt = lax.fori_loop(0, SEG // LN, cb, 0)
                    for k in range(BK // LN):
                        cld[pl.ds(cnt + k * LN, LN)] = jnp.full((LN,), C, i32)
                        cpos[pl.ds(cnt + k * LN, LN)] = jnp.full(
                            (LN,), base, i32)
                    kb = (cnt + BK - 1) // BK

                    def blk(k, _):
                        boff = k * BK
                        pltpu.sync_copy(src_hbm.at[cpos.at[pl.ds(boff, BK)]],
                                        srcb)
                        pltpu.sync_copy(h_hbm.at[srcb], hrows)
                        pltpu.sync_copy(invout_hbm.at[srcb], scal_v)
                        if use_rel:
                            pltpu.sync_copy(
                                rel_hbm.at[cpos.at[pl.ds(boff, BK)]], relb)
                            pltpu.sync_copy(relT_sh.at[relb], rrows)

                        def rowm(gr, _):
                            scv = scal_v[pl.ds(gr * LN, LN)]
                            for r16 in range(LN):
                                r = gr * LN + r16
                                sc = jnp.broadcast_to(scv[r16], (LN,))
                                for j in range(8):
                                    x = hrows[r, pl.ds(j * LN, LN)] * sc
                                    if use_rel:
                                        x = x * rrows[r, pl.ds(j * LN, LN)]
                                    hrows[r, pl.ds(j * LN, LN)] = x
                            return 0

                        lax.fori_loop(0, BK // LN, rowm, 0)
                        for v in range(BK // LN):
                            idxb[pl.ds(v * LN, LN)] = (
                                cld[pl.ds(boff + v * LN, LN)])
                        pltpu.sync_copy(hrows, agg_sh.at[idxb], add=True)
                        return 0

                    lax.fori_loop(0, kb, blk, 0)
                    return 0

                lax.fori_loop(0, NSEG, seg, 0)

            plsc.subcore_barrier()

            @pl.when(active)
            def _():
                def eb(k, _):
                    row0 = lo + s * EROWS + k * BK
                    pltpu.sync_copy(agg_sh.at[pl.ds(s * EROWS + k * BK, BK)],
                                    hrows)
                    pltpu.sync_copy(invin_hbm.at[pl.ds(row0, BK)], scal_v)
                    if residual:
                        for q in range(BK // LN):
                            hq = jnp.minimum(row0 + q * LN, Nh - LN)
                            pltpu.sync_copy(h_hbm.at[pl.ds(hq, LN)],
                                            rrows.at[pl.ds(q * LN, LN)])

                    def rowm(gr, _):
                        scv = scal_v[pl.ds(gr * LN, LN)]
                        for r16 in range(LN):
                            r = gr * LN + r16
                            iv = jnp.broadcast_to(scv[r16], (LN,))
                            for j in range(8):
                                x = hrows[r, pl.ds(j * LN, LN)] * iv
                                if residual:
                                    x = x + rrows[r, pl.ds(j * LN, LN)]
                                hrows[r, pl.ds(j * LN, LN)] = x
                        return 0

                    lax.fori_loop(0, BK // LN, rowm, 0)
                    pltpu.sync_copy(hrows, out_hbm.at[pl.ds(row0, BK)])
                    return 0

                lax.fori_loop(0, EROWS // BK, eb, 0)

            return 0

        lax.fori_loop(0, NCH, chunk, 0)

    return gcn_kernel


def _make_seed_prep_kernel(BWT, BWG, SLT, SLG):
    B = 4096
    PT = B // NS
    CT, CG = BWT * 32, BWG * 32
    assert (BWT // NS) % LN == 0 and (BWG // NS) % LN == 0
    assert (CT // NS) % LN == 0 and (CG // NS) % LN == 0

    @functools.partial(
        pl.kernel,
        out_type=(
            jax.ShapeDtypeStruct((BWT,), i32),
            jax.ShapeDtypeStruct((BWG,), i32),
            jax.ShapeDtypeStruct((SLT,), i32),
            jax.ShapeDtypeStruct((SLG,), i32),
        ),
        mesh=_mesh(),
        compiler_params=pltpu.CompilerParams(needs_layout_passes=False),
        scratch_types=[
            pltpu.VMEM_SHARED((CT,), f32),
            pltpu.VMEM_SHARED((CG,), f32),
            pltpu.VMEM_SHARED((SLT,), i32),
            pltpu.VMEM_SHARED((SLG,), i32),
            pltpu.VMEM((CT // NS,), f32),
            pltpu.VMEM((PT,), i32),
            [pltpu.VMEM((128,), i32) for _ in range(2)],
            [pltpu.VMEM((128,), i32) for _ in range(2)],
            pltpu.VMEM((128,), f32),
            pltpu.VMEM((BWT // NS,), i32),
            pltpu.VMEM((512,), f32),
            pltpu.VMEM((SLT // NS // 2,), i32),
            pltpu.SemaphoreType.DMA,
        ],
    )
    def prep_kernel(seed_hbm, bt_hbm, bg_hbm, st_hbm, sg_hbm,
                    cntT_sh, cntG_sh, slT_sh, slG_sh, zeros_v, seed_v,
                    idxb_v, val_v, ones_v, pack_v, cbuf_v, sbounce_v, sem):
        c = lax.axis_index("c")
        s = lax.axis_index("s")
        ZB = CT // NS
        _zero_1d(zeros_v, ZB)

        def ob(idx, _):
            ones_v[pl.ds(idx * LN, LN)] = jnp.ones((LN,), f32)
            return 0

        lax.fori_loop(0, 128 // LN, ob, 0)
        pltpu.sync_copy(zeros_v, cntT_sh.at[pl.ds(s * ZB, ZB)])
        pltpu.sync_copy(zeros_v.at[pl.ds(0, CG // NS)],
                        cntG_sh.at[pl.ds(s * (CG // NS), CG // NS)])
        plsc.subcore_barrier()

        base = s * PT
        pltpu.sync_copy(seed_hbm.at[pl.ds(base, PT)], seed_v)
        for half in range(PT // 128):
            for v in range(128 // LN):
                sv = seed_v[pl.ds(half * 128 + v * LN, LN)]
                idxb_v[half][pl.ds(v * LN, LN)] = sv
                pos = (jnp.full((LN,), base + half * 128 + v * LN, i32)
                       + lax.iota(i32, LN))
                val_v[half][pl.ds(v * LN, LN)] = pos
            pltpu.async_copy(ones_v, cntT_sh.at[idxb_v[half]], sem, add=True)
            pltpu.make_async_copy(ones_v, cntT_sh.at[idxb_v[half]], sem).wait()
            pltpu.sync_copy(val_v[half], slT_sh.at[idxb_v[half]])
        for half in range(PT // 128):
            for v in range(128 // LN):
                sv = seed_v[pl.ds(half * 128 + v * LN, LN)]
                idxb_v[half][pl.ds(v * LN, LN)] = (
                    lax.shift_right_logical(sv, 2))
            pltpu.async_copy(ones_v, cntG_sh.at[idxb_v[half]], sem, add=True)
            pltpu.make_async_copy(ones_v, cntG_sh.at[idxb_v[half]], sem).wait()
            pltpu.sync_copy(val_v[half], slG_sh.at[idxb_v[half]])
        plsc.subcore_barrier()

        SPT = SLT // NS
        HB = SPT // 2
        for z in range(2):
            pltpu.sync_copy(slT_sh.at[pl.ds(s * SPT + z * HB, HB)], sbounce_v)
            pltpu.sync_copy(sbounce_v, st_hbm.at[pl.ds(s * SPT + z * HB, HB)])
        SPG = SLG // NS
        pltpu.sync_copy(slG_sh.at[pl.ds(s * SPG, SPG)],
                        sbounce_v.at[pl.ds(0, SPG)])
        pltpu.sync_copy(sbounce_v.at[pl.ds(0, SPG)],
                        sg_hbm.at[pl.ds(s * SPG, SPG)])

        powers = lax.shift_left(jnp.ones((LN,), i32), lax.iota(i32, LN))

        def pack_loop(cnt_sh, b_hbm, bw):
            span = bw // NS

            def grp(g, _):
                acc = jnp.zeros((LN,), i32)
                pltpu.sync_copy(
                    cnt_sh.at[pl.ds((s * span + g * LN) * 32, 512)], cbuf_v)
                for j in range(LN):
                    clo = cbuf_v[pl.ds(j * 32, LN)]
                    chi = cbuf_v[pl.ds(j * 32 + LN, LN)]
                    wlo = jnp.sum(jnp.where(clo > 0.0, powers, 0))
                    whi = jnp.sum(jnp.where(chi > 0.0, powers, 0))
                    wrd = wlo + lax.shift_left(whi, 16)
                    acc = jnp.where(lax.iota(i32, LN) == j,
                                    jnp.broadcast_to(wrd, (LN,)), acc)
                pack_v[pl.ds(g * LN, LN)] = acc
                return 0

            lax.fori_loop(0, span // LN, grp, 0)
            pltpu.sync_copy(pack_v.at[pl.ds(0, span)],
                            b_hbm.at[pl.ds(s * span, span)])

        pack_loop(cntT_sh, bt_hbm, BWT)
        pack_loop(cntG_sh, bg_hbm, BWG)

    return prep_kernel


def _make_pruned_agg_kernel(N, BW, ER, use_rel):
    B = 4096
    STRIPE = ER // (NC * NS)
    BK = 64
    CB = STRIPE + BK
    EROWS = B // NS

    scratch = [
        pltpu.VMEM_SHARED((B + 8, 128), f32),
        pltpu.VMEM((BW,), i32),
        pltpu.VMEM((STRIPE,), i32),
        pltpu.VMEM((CB,), i32),
        pltpu.VMEM((BK,), i32),
        pltpu.VMEM((BK,), i32),
        pltpu.VMEM((BK,), i32),
        pltpu.VMEM((BK, 128), f32),
        pltpu.VMEM((BK,), f32),
        pltpu.VMEM((16, 128), f32),
        pltpu.SemaphoreType.DMA,
        pltpu.SemaphoreType.DMA,
    ]
    if use_rel:
        scratch += [
            pltpu.VMEM_SHARED((1024, 128), f32),
            pltpu.VMEM((BK,), i32),
            pltpu.VMEM((BK, 128), f32),
        ]

    @functools.partial(
        pl.kernel,
        out_type=jax.ShapeDtypeStruct((2, B + 8, 128), f32),
        mesh=_mesh(),
        compiler_params=pltpu.CompilerParams(needs_layout_passes=False),
        scratch_types=scratch,
    )
    def pagg_kernel(*args):
        if use_rel:
            (h_hbm, src_hbm, dst_hbm, rel_hbm, invout_hbm, relT_hbm,
             bmap_hbm, slot_hbm, out_hbm,
             agg_sh, bmapv, dstrip, cpos, srcb, dstb, slotb, hrows, scal_v,
             zeros_v, semi, semh, relT_sh, relb, rrows) = args
        else:
            (h_hbm, src_hbm, dst_hbm, invout_hbm,
             bmap_hbm, slot_hbm, out_hbm,
             agg_sh, bmapv, dstrip, cpos, srcb, dstb, slotb, hrows, scal_v,
             zeros_v, semi, semh) = args

        c = lax.axis_index("c")
        s = lax.axis_index("s")
        wid = s * NC + c

        def zb(r, _):
            for j in range(8):
                zeros_v[r, pl.ds(j * LN, LN)] = jnp.zeros((LN,), f32)
            return 0

        lax.fori_loop(0, 16, zb, 0)

        def azb(k, _):
            pltpu.sync_copy(zeros_v, agg_sh.at[pl.ds(s * EROWS + k * 16, 16)])
            return 0

        lax.fori_loop(0, EROWS // 16, azb, 0)

        if use_rel:
            pltpu.sync_copy(relT_hbm.at[pl.ds(s * 64, 64)],
                            relT_sh.at[pl.ds(s * 64, 64)])
        pltpu.sync_copy(bmap_hbm, bmapv)
        base = wid * STRIPE
        pltpu.sync_copy(dst_hbm.at[pl.ds(base, STRIPE)], dstrip)
        plsc.subcore_barrier()

        def cb(v, cnt):
            off = v * LN
            d = dstrip[pl.ds(off, LN)]
            wi = jnp.minimum(lax.shift_right_logical(d, 5), BW - 1)
            w = plsc.load_gather(bmapv, [wi])
            bit = lax.shift_right_logical(w, d & 31) & 1
            m = (bit == 1) & (d < N)
            pos = jnp.full((LN,), base + off, i32) + lax.iota(i32, LN)
            plsc.store_compressed(cpos.at[pl.ds(cnt, LN)], pos, mask=m)
            return cnt + jnp.sum(jnp.where(m, 1, 0))

        cnt = lax.fori_loop(0, STRIPE // LN, cb, 0)
        for k in range(BK // LN):
            cpos[pl.ds(cnt + k * LN, LN)] = jnp.full((LN,), ER - 1, i32)
        kb = (cnt + BK - 1) // BK

        def blk(k, _):
            boff = k * BK
            pltpu.sync_copy(src_hbm.at[cpos.at[pl.ds(boff, BK)]], srcb)
            pltpu.sync_copy(dst_hbm.at[cpos.at[pl.ds(boff, BK)]], dstb)
            if use_rel:
                pltpu.sync_copy(rel_hbm.at[cpos.at[pl.ds(boff, BK)]], relb)
            pltpu.sync_copy(h_hbm.at[srcb], hrows)
            pltpu.sync_copy(invout_hbm.at[srcb], scal_v)
            pltpu.sync_copy(slot_hbm.at[dstb], slotb)
            if use_rel:
                pltpu.sync_copy(relT_sh.at[relb], rrows)
            for v in range(BK // LN):
                lane = (jnp.full((LN,), boff + v * LN, i32)
                        + lax.iota(i32, LN))
                sl = slotb[pl.ds(v * LN, LN)]
                slotb[pl.ds(v * LN, LN)] = jnp.where(lane < cnt, sl, B)

            def rowm(gr, _):
                scv = scal_v[pl.ds(gr * LN, LN)]
                for r16 in range(LN):
                    r = gr * LN + r16
                    sc = jnp.broadcast_to(scv[r16], (LN,))
                    for j in range(8):
                        x = hrows[r, pl.ds(j * LN, LN)] * sc
                        if use_rel:
                            x = x * rrows[r, pl.ds(j * LN, LN)]
                        hrows[r, pl.ds(j * LN, LN)] = x
                return 0

            lax.fori_loop(0, BK // LN, rowm, 0)
            pltpu.sync_copy(hrows, agg_sh.at[slotb], add=True)
            return 0

        lax.fori_loop(0, kb, blk, 0)
        plsc.subcore_barrier()

        def wo(k, _):
            r0 = s * EROWS + k * BK
            pltpu.sync_copy(agg_sh.at[pl.ds(r0, BK)], hrows)
            pltpu.sync_copy(hrows, out_hbm.at[c, pl.ds(r0, BK)])
            return 0

        lax.fori_loop(0, EROWS // BK, wo, 0)

    return pagg_kernel


def _make_final_kernel():
    B = 4096
    PW = B // (NC * NS)

    @functools.partial(
        pl.kernel,
        out_type=(
            jax.ShapeDtypeStruct((B, 128), f32),
            jax.ShapeDtypeStruct((B, 128), f32),
        ),
        mesh=_mesh(),
        compiler_params=pltpu.CompilerParams(needs_layout_passes=False),
        scratch_types=[
            pltpu.VMEM((PW,), i32),
            pltpu.VMEM((PW,), i32),
            pltpu.VMEM((PW,), f32),
            pltpu.VMEM((PW, 128), f32),
            pltpu.VMEM((PW, 128), f32),
            pltpu.VMEM((PW, 128), f32),
            pltpu.VMEM((PW, 128), f32),
        ],
    )
    def final_kernel(ht_hbm, hg_hbm, at0_hbm, at1_hbm, ag0_hbm, ag1_hbm,
                     it_hbm, ig_hbm, st_hbm, sg_hbm,
                     er_hbm, ee_hbm, seed_hbm, rb_hbm,
                     n_hbm, e_hbm, ia_v, ib_v, sc_v, a_v, b_v, h_v, n_v):
        c = lax.axis_index("c")
        s = lax.axis_index("s")
        wid = s * NC + c
        base = wid * PW

        pltpu.sync_copy(seed_hbm.at[pl.ds(base, PW)], ia_v)

        def half_path(agg0, agg1, inv_hbm, slot_hbm, hop1_hbm, acc):
            pltpu.sync_copy(inv_hbm.at[ia_v], sc_v)
            pltpu.sync_copy(slot_hbm.at[ia_v], ib_v)
            pltpu.sync_copy(agg0.at[ib_v], a_v)
            pltpu.sync_copy(agg1.at[ib_v], b_v)
            pltpu.sync_copy(hop1_hbm.at[ia_v], h_v)

            def rowm(gr, _):
                scv = sc_v[pl.ds(gr * LN, LN)]
                for r16 in range(LN):
                    r = gr * LN + r16
                    iv = jnp.broadcast_to(scv[r16], (LN,))
                    for j in range(8):
                        x = ((a_v[r, pl.ds(j * LN, LN)]
                              + b_v[r, pl.ds(j * LN, LN)]) * iv
                             + h_v[r, pl.ds(j * LN, LN)]) * 0.5
                        if acc:
                            x = x + n_v[r, pl.ds(j * LN, LN)]
                        n_v[r, pl.ds(j * LN, LN)] = x
                return 0

            lax.fori_loop(0, PW // LN, rowm, 0)

        half_path(at0_hbm, at1_hbm, it_hbm, st_hbm, ht_hbm, False)

        def ob(idx, _):
            sv = ia_v[pl.ds(idx * LN, LN)]
            ia_v[pl.ds(idx * LN, LN)] = lax.shift_right_logical(sv, 2)
            return 0

        lax.fori_loop(0, PW // LN, ob, 0)
        half_path(ag0_hbm, ag1_hbm, ig_hbm, sg_hbm, hg_hbm, True)
        pltpu.sync_copy(n_v, n_hbm.at[pl.ds(base, PW)])

        pltpu.sync_copy(rb_hbm.at[pl.ds(base, PW)], ia_v)
        pltpu.sync_copy(er_hbm.at[ia_v], a_v)
        pltpu.sync_copy(ee_hbm.at[ia_v], b_v)

        def rowe(r, _):
            for j in range(8):
                a_v[r, pl.ds(j * LN, LN)] = (
                    a_v[r, pl.ds(j * LN, LN)] * 0.5
                    + b_v[r, pl.ds(j * LN, LN)] * 0.5)
            return 0

        lax.fori_loop(0, PW, rowe, 0)
        pltpu.sync_copy(a_v, e_hbm.at[pl.ds(base, PW)])

    return final_kernel


_NT, _NG, _NRL = 200000, 50000, 1000
_C = 8192
_NR_T, _NCH_T = 204800, 25
_NR_G, _NCH_G = 57344, 7
_NR_R, _NCH_R = 1024, 1
_ER_BIG, _ER_REL = 409600, 16384
_BWT, _BWG = 6400, 2048
_SLT, _SLG = 206848, 59392

_deg_t = _make_deg_kernel(_NR_T, _NR_T + 2048, _ER_BIG)
_deg_g = _make_deg_kernel(_NR_G, _NR_G + 2048, _ER_BIG)
_deg_r = _make_deg_kernel(_NR_R, _NR_R + 2048, _ER_REL)
_gcn_t = _make_gcn_kernel(_NT, _NR_T, _C, _NCH_T, _ER_BIG, True, True)
_gcn_g = _make_gcn_kernel(_NG, _NR_G, _C, _NCH_G, _ER_BIG, True, True)
_gcn_r = _make_gcn_kernel(_NR_R, _NR_R, _NR_R, _NCH_R, _ER_REL, False, False)
_prep = _make_seed_prep_kernel(_BWT, _BWG, _SLT, _SLG)
_pagg_t = _make_pruned_agg_kernel(_NT, _BWT, _ER_BIG, True)
_pagg_g = _make_pruned_agg_kernel(_NG, _BWG, _ER_BIG, True)
_final = _make_final_kernel()


def _pad_edges(src, dst, rel, er, dummy_src, dummy_dst):
    pe = er - src.shape[0]
    out = (jnp.concatenate([src, jnp.full((pe,), dummy_src, i32)]),
           jnp.concatenate([dst, jnp.full((pe,), dummy_dst, i32)]))
    if rel is not None:
        out += (jnp.concatenate([rel, jnp.zeros((pe,), i32)]),)
    return out


def kernel(g_edge_index, g_edge_rel, glob_edge_index, glob_edge_rel,
           rel_edge_index, seed_nodes, relation_batch, neighbor_batch_size,
           node_emb, global_emb, edge_emb):
    del neighbor_batch_size
    tsrc, tdst, trel = _pad_edges(
        g_edge_index[0], g_edge_index[1], g_edge_rel, _ER_BIG, _NT, _NR_T)
    gsrc, gdst, grel = _pad_edges(
        glob_edge_index[0], glob_edge_index[1], glob_edge_rel, _ER_BIG,
        _NG, _NR_G)
    rsrc, rdst = _pad_edges(
        rel_edge_index[0], rel_edge_index[1], None, _ER_REL, _NRL, _NRL)
    ee_pad = jnp.concatenate(
        [edge_emb, jnp.zeros((_NR_R - _NRL, 128), f32)], axis=0)

    t_inv = _deg_t(jnp.stack([tsrc, tdst]))
    g_inv = _deg_g(jnp.stack([gsrc, gdst]))
    r_inv = _deg_r(jnp.stack([rsrc, rdst]))
    t_io, t_ii = t_inv[0], t_inv[1]
    g_io, g_ii = g_inv[0], g_inv[1]
    r_io, r_ii = r_inv[0], r_inv[1]

    bmap_t, bmap_g, slot_t, slot_g = _prep(seed_nodes)

    ht1 = _gcn_t(node_emb, tsrc, tdst, trel, t_io, t_ii, ee_pad)
    hg1 = _gcn_g(global_emb, gsrc, gdst, grel, g_io, g_ii, ee_pad)
    at = _pagg_t(ht1, tsrc, tdst, trel, t_io, ee_pad, bmap_t, slot_t)
    ag = _pagg_g(hg1, gsrc, gdst, grel, g_io, ee_pad, bmap_g, slot_g)
    er = _gcn_r(ee_pad, rsrc, rdst, r_io, r_ii)

    n, e = _final(ht1, hg1, at[0], at[1], ag[0], ag[1], t_ii, g_ii,
                  slot_t, slot_g, er, ee_pad, seed_nodes, relation_batch)
    return n, e


# speedup vs baseline: 3.2485x; 1.1913x over previous
import functools

import jax
import jax.numpy as jnp
from jax import lax
from jax.experimental import pallas as pl
from jax.experimental.pallas import tpu as pltpu
from jax.experimental.pallas import tpu_sc as plsc

NC, NS, LN = 2, 16, 16
f32 = jnp.float32
i32 = jnp.int32


def _mesh():
    return plsc.VectorSubcoreMesh(
        core_axis_name="c", subcore_axis_name="s", num_cores=NC, num_subcores=NS
    )


def _rsqrt16(x):
    y = 1.0 / x
    for _ in range(28):
        y = y * (1.5 - 0.5 * x * y * y)
    return y


def _zero_1d(ref, n):
    def body(idx, _):
        ref[pl.ds(idx * LN, LN)] = jnp.zeros((LN,), f32)
        return 0

    lax.fori_loop(0, n // LN, body, 0)


def _make_deg_kernel(NR, NRT, ER):
    STRIPE = ER // NS
    NB = STRIPE // 128
    assert NB % 8 == 0 or NB == 8
    ZSPAN = NRT // NS
    ZB = ZSPAN if ZSPAN <= 6464 else ZSPAN // 2
    assert ZSPAN % ZB == 0 and ZB % LN == 0
    WSPAN = NR // NS
    WCH = 6400 if WSPAN % 6400 == 0 else WSPAN
    assert WSPAN % WCH == 0 and WCH % LN == 0

    @functools.partial(
        pl.kernel,
        out_type=jax.ShapeDtypeStruct((2, NR), f32),
        mesh=_mesh(),
        compiler_params=pltpu.CompilerParams(needs_layout_passes=False),
        scratch_types=[
            pltpu.VMEM_SHARED((NRT,), f32),
            pltpu.VMEM((STRIPE,), i32),
            [pltpu.VMEM((128,), i32) for _ in range(8)],
            pltpu.VMEM((128,), f32),
            pltpu.VMEM((ZB,), f32),
            pltpu.VMEM((WCH,), f32),
            pltpu.SemaphoreType.DMA,
        ],
    )
    def deg_kernel(edges_hbm, inv_hbm,
                   deg_sh, idx1_v, idxb_v, ones_v, zeros_v, val_v, sem):
        c = lax.axis_index("c")
        s = lax.axis_index("s")

        _zero_1d(zeros_v, ZB)

        def ob(idx, _):
            ones_v[pl.ds(idx * LN, LN)] = jnp.ones((LN,), f32)
            return 0

        lax.fori_loop(0, 128 // LN, ob, 0)

        for z in range(ZSPAN // ZB):
            pltpu.sync_copy(zeros_v, deg_sh.at[pl.ds(s * ZSPAN + z * ZB, ZB)])
        plsc.subcore_barrier()

        base = s * STRIPE
        pltpu.sync_copy(edges_hbm.at[c, pl.ds(base, STRIPE)], idx1_v)

        def grp(g, _):
            for k in range(8):
                b = g * 8 + k
                for v in range(128 // LN):
                    idxb_v[k][pl.ds(v * LN, LN)] = (
                        idx1_v[pl.ds(b * 128 + v * LN, LN)])
                pltpu.async_copy(ones_v, deg_sh.at[idxb_v[k]], sem, add=True)
            for k in range(8):
                pltpu.make_async_copy(ones_v, deg_sh.at[idxb_v[k]], sem).wait()
            return 0

        lax.fori_loop(0, NB // 8, grp, 0)
        plsc.subcore_barrier()

        def wo(w, _):
            off = s * WSPAN + w * WCH
            pltpu.sync_copy(deg_sh.at[pl.ds(off, WCH)], val_v)

            def rb(idx, _):
                d = val_v[pl.ds(idx * LN, LN)]
                val_v[pl.ds(idx * LN, LN)] = _rsqrt16(jnp.maximum(d, 1.0))
                return 0

            lax.fori_loop(0, WCH // LN, rb, 0)
            pltpu.sync_copy(val_v, inv_hbm.at[c, pl.ds(off, WCH)])
            return 0

        lax.fori_loop(0, WSPAN // WCH, wo, 0)

    return deg_kernel


def _make_gcn_kernel(Nh, NR, C, NCH, ER, use_rel, residual):
    STRIPE = ER // NS
    SEG = 3200 if STRIPE % 3200 == 0 else STRIPE
    NSEG = STRIPE // SEG
    BK = 64
    CB = SEG + BK
    EROWS = C // NS
    assert C * NCH == NR and EROWS % BK == 0 and SEG % LN == 0

    scratch = [
        pltpu.VMEM_SHARED((C + 8, 128), f32),
        pltpu.VMEM((STRIPE,), i32),
        pltpu.VMEM((CB,), i32),
        pltpu.VMEM((CB,), i32),
        pltpu.VMEM((BK,), i32),
        pltpu.VMEM((BK,), i32),
        pltpu.VMEM((BK, 128), f32),
        pltpu.VMEM((BK,), f32),
        pltpu.VMEM((16, 128), f32),
    ]
    if use_rel:
        scratch += [
            pltpu.VMEM_SHARED((1024, 128), f32),
            pltpu.VMEM((BK,), i32),
            pltpu.VMEM((BK, 128), f32),
        ]
    elif residual:
        scratch += [pltpu.VMEM((BK, 128), f32)]

    @functools.partial(
        pl.kernel,
        out_type=jax.ShapeDtypeStruct((NR, 128), f32),
        mesh=_mesh(),
        compiler_params=pltpu.CompilerParams(needs_layout_passes=False),
        scratch_types=scratch,
    )
    def gcn_kernel(*args):
        if use_rel:
            (h_hbm, src_hbm, dst_hbm, rel_hbm, invout_hbm, invin_hbm,
             relT_hbm, out_hbm,
             agg_sh, dstrip, cld, cpos, idxb, srcb, hrows,
             scal_v, zeros_v, relT_sh, relb, rrows) = args
        elif residual:
            (h_hbm, src_hbm, dst_hbm, invout_hbm, invin_hbm, out_hbm,
             agg_sh, dstrip, cld, cpos, idxb, srcb, hrows,
             scal_v, zeros_v, rrows) = args
        else:
            (h_hbm, src_hbm, dst_hbm, invout_hbm, invin_hbm, out_hbm,
             agg_sh, dstrip, cld, cpos, idxb, srcb, hrows,
             scal_v, zeros_v) = args

        c = lax.axis_index("c")
        s = lax.axis_index("s")

        def zb(r, _):
            for j in range(8):
                zeros_v[r, pl.ds(j * LN, LN)] = jnp.zeros((LN,), f32)
            return 0

        lax.fori_loop(0, 16, zb, 0)

        if use_rel:
            pltpu.sync_copy(relT_hbm.at[pl.ds(s * 64, 64)],
                            relT_sh.at[pl.ds(s * 64, 64)])
        base = s * STRIPE
        pltpu.sync_copy(dst_hbm.at[pl.ds(base, STRIPE)], dstrip)
        plsc.subcore_barrier()

        def chunk(g, _):
            lo = g * C
            active = lax.rem(g, 2) == c

            @pl.when(active)
            def _():
                def azb(k, _):
                    pltpu.sync_copy(
                        zeros_v, agg_sh.at[pl.ds(s * EROWS + k * 16, 16)])
                    return 0

                lax.fori_loop(0, EROWS // 16, azb, 0)

            plsc.subcore_barrier()

            @pl.when(active)
            def _():
                def seg(t, _):
                    def cb(v, cnt):
                        off = t * SEG + v * LN
                        d = dstrip[pl.ds(off, LN)]
                        m = (d >= lo) & (d < lo + C)
                        plsc.store_compressed(cld.at[pl.ds(cnt, LN)], d - lo,
                                              mask=m)
                        pos = jnp.full((LN,), base + off, i32) + lax.iota(i32, LN)
                        plsc.store_compressed(cpos.at[pl.ds(cnt, LN)], pos,
                                              mask=m)
                        return cnt + jnp.sum(jnp.where(m, 1, 0))

                    cnt = lax.fori_loop(0, SEG // LN, cb, 0)
                    for k in range(BK // LN):
                        cld[pl.ds(cnt + k * LN, LN)] = jnp.full((LN,), C, i32)
                        cpos[pl.ds(cnt + k * LN, LN)] = jnp.full(
                            (LN,), base, i32)
                    kb = (cnt + BK - 1) // BK

                    def blk(k, _):
                        boff = k * BK
                        pltpu.sync_copy(src_hbm.at[cpos.at[pl.ds(boff, BK)]],
                                        srcb)
                        pltpu.sync_copy(h_hbm.at[srcb], hrows)
                        pltpu.sync_copy(invout_hbm.at[srcb], scal_v)
                        if use_rel:
                            pltpu.sync_copy(
                                rel_hbm.at[cpos.at[pl.ds(boff, BK)]], relb)
                            pltpu.sync_copy(relT_sh.at[relb], rrows)

                        def rowm(gr, _):
                            scv = scal_v[pl.ds(gr * LN, LN)]
                            for r16 in range(LN):
                                r = gr * LN + r16
                                sc = jnp.broadcast_to(scv[r16], (LN,))
                                for j in range(8):
                                    x = hrows[r, pl.ds(j * LN, LN)] * sc
                                    if use_rel:
                                        x = x * rrows[r, pl.ds(j * LN, LN)]
                                    hrows[r, pl.ds(j * LN, LN)] = x
                            return 0

                        lax.fori_loop(0, BK // LN, rowm, 0)
                        for v in range(BK // LN):
                            idxb[pl.ds(v * LN, LN)] = (
                                cld[pl.ds(boff + v * LN, LN)])
                        pltpu.sync_copy(hrows, agg_sh.at[idxb], add=True)
                        return 0

                    lax.fori_loop(0, kb, blk, 0)
                    return 0

                lax.fori_loop(0, NSEG, seg, 0)

            plsc.subcore_barrier()

            @pl.when(active)
            def _():
                def eb(k, _):
                    row0 = lo + s * EROWS + k * BK
                    pltpu.sync_copy(agg_sh.at[pl.ds(s * EROWS + k * BK, BK)],
                                    hrows)
                    pltpu.sync_copy(invin_hbm.at[pl.ds(row0, BK)], scal_v)
                    if residual:
                        for q in range(BK // LN):
                            hq = jnp.minimum(row0 + q * LN, Nh - LN)
                            pltpu.sync_copy(h_hbm.at[pl.ds(hq, LN)],
                                            rrows.at[pl.ds(q * LN, LN)])

                    def rowm(gr, _):
                        scv = scal_v[pl.ds(gr * LN, LN)]
                        for r16 in range(LN):
                            r = gr * LN + r16
                            iv = jnp.broadcast_to(scv[r16], (LN,))
                            for j in range(8):
                                x = hrows[r, pl.ds(j * LN, LN)] * iv
                                if residual:
                                    x = x + rrows[r, pl.ds(j * LN, LN)]
                                hrows[r, pl.ds(j * LN, LN)] = x
                        return 0

                    lax.fori_loop(0, BK // LN, rowm, 0)
                    pltpu.sync_copy(hrows, out_hbm.at[pl.ds(row0, BK)])
                    return 0

                lax.fori_loop(0, EROWS // BK, eb, 0)

            return 0

        lax.fori_loop(0, NCH, chunk, 0)

    return gcn_kernel


def _make_seed_prep_kernel(BWT, BWG, SLT, SLG):
    B = 4096
    PT = B // NS
    CT, CG = BWT * 32, BWG * 32
    assert (BWT // NS) % LN == 0 and (BWG // NS) % LN == 0
    assert (CT // NS) % LN == 0 and (CG // NS) % LN == 0

    @functools.partial(
        pl.kernel,
        out_type=(
            jax.ShapeDtypeStruct((BWT,), i32),
            jax.ShapeDtypeStruct((BWG,), i32),
            jax.ShapeDtypeStruct((SLT,), i32),
            jax.ShapeDtypeStruct((SLG,), i32),
        ),
        mesh=_mesh(),
        compiler_params=pltpu.CompilerParams(needs_layout_passes=False),
        scratch_types=[
            pltpu.VMEM_SHARED((CT,), f32),
            pltpu.VMEM_SHARED((CG,), f32),
            pltpu.VMEM_SHARED((SLT,), i32),
            pltpu.VMEM_SHARED((SLG,), i32),
            pltpu.VMEM((CT // NS,), f32),
            pltpu.VMEM((PT,), i32),
            [pltpu.VMEM((128,), i32) for _ in range(2)],
            [pltpu.VMEM((128,), i32) for _ in range(2)],
            pltpu.VMEM((128,), f32),
            pltpu.VMEM((BWT // NS,), i32),
            pltpu.VMEM((512,), f32),
            pltpu.VMEM((SLT // NS // 2,), i32),
            pltpu.SemaphoreType.DMA,
        ],
    )
    def prep_kernel(seed_hbm, bt_hbm, bg_hbm, st_hbm, sg_hbm,
                    cntT_sh, cntG_sh, slT_sh, slG_sh, zeros_v, seed_v,
                    idxb_v, val_v, ones_v, pack_v, cbuf_v, sbounce_v, sem):
        c = lax.axis_index("c")
        s = lax.axis_index("s")
        ZB = CT // NS
        _zero_1d(zeros_v, ZB)

        def ob(idx, _):
            ones_v[pl.ds(idx * LN, LN)] = jnp.ones((LN,), f32)
            return 0

        lax.fori_loop(0, 128 // LN, ob, 0)
        pltpu.sync_copy(zeros_v, cntT_sh.at[pl.ds(s * ZB, ZB)])
        pltpu.sync_copy(zeros_v.at[pl.ds(0, CG // NS)],
                        cntG_sh.at[pl.ds(s * (CG // NS), CG // NS)])
        plsc.subcore_barrier()

        base = s * PT
        pltpu.sync_copy(seed_hbm.at[pl.ds(base, PT)], seed_v)
        for half in range(PT // 128):
            for v in range(128 // LN):
                sv = seed_v[pl.ds(half * 128 + v * LN, LN)]
                idxb_v[half][pl.ds(v * LN, LN)] = sv
                pos = (jnp.full((LN,), base + half * 128 + v * LN, i32)
                       + lax.iota(i32, LN))
                val_v[half][pl.ds(v * LN, LN)] = pos
            pltpu.async_copy(ones_v, cntT_sh.at[idxb_v[half]], sem, add=True)
            pltpu.make_async_copy(ones_v, cntT_sh.at[idxb_v[half]], sem).wait()
            pltpu.sync_copy(val_v[half], slT_sh.at[idxb_v[half]])
        for half in range(PT // 128):
            for v in range(128 // LN):
                sv = seed_v[pl.ds(half * 128 + v * LN, LN)]
                idxb_v[half][pl.ds(v * LN, LN)] = (
                    lax.shift_right_logical(sv, 2))
            pltpu.async_copy(ones_v, cntG_sh.at[idxb_v[half]], sem, add=True)
            pltpu.make_async_copy(ones_v, cntG_sh.at[idxb_v[half]], sem).wait()
            pltpu.sync_copy(val_v[half], slG_sh.at[idxb_v[half]])
        plsc.subcore_barrier()

        SPT = SLT // NS
        HB = SPT // 2
        for z in range(2):
            pltpu.sync_copy(slT_sh.at[pl.ds(s * SPT + z * HB, HB)], sbounce_v)
            pltpu.sync_copy(sbounce_v, st_hbm.at[pl.ds(s * SPT + z * HB, HB)])
        SPG = SLG // NS
        pltpu.sync_copy(slG_sh.at[pl.ds(s * SPG, SPG)],
                        sbounce_v.at[pl.ds(0, SPG)])
        pltpu.sync_copy(sbounce_v.at[pl.ds(0, SPG)],
                        sg_hbm.at[pl.ds(s * SPG, SPG)])

        powers = lax.shift_left(jnp.ones((LN,), i32), lax.iota(i32, LN))

        def pack_loop(cnt_sh, b_hbm, bw):
            span = bw // NS

            def grp(g, _):
                acc = jnp.zeros((LN,), i32)
                pltpu.sync_copy(
                    cnt_sh.at[pl.ds((s * span + g * LN) * 32, 512)], cbuf_v)
                for j in range(LN):
                    clo = cbuf_v[pl.ds(j * 32, LN)]
                    chi = cbuf_v[pl.ds(j * 32 + LN, LN)]
                    wlo = jnp.sum(jnp.where(clo > 0.0, powers, 0))
                    whi = jnp.sum(jnp.where(chi > 0.0, powers, 0))
                    wrd = wlo + lax.shift_left(whi, 16)
                    acc = jnp.where(lax.iota(i32, LN) == j,
                                    jnp.broadcast_to(wrd, (LN,)), acc)
                pack_v[pl.ds(g * LN, LN)] = acc
                return 0

            lax.fori_loop(0, span // LN, grp, 0)
            pltpu.sync_copy(pack_v.at[pl.ds(0, span)],
                            b_hbm.at[pl.ds(s * span, span)])

        pack_loop(cntT_sh, bt_hbm, BWT)
        pack_loop(cntG_sh, bg_hbm, BWG)

    return prep_kernel


def _make_pruned_agg_kernel(N, BW, ER, use_rel):
    B = 4096
    STRIPE = ER // (NC * NS)
    BK = 64
    CB = STRIPE + BK
    EROWS = B // NS

    scratch = [
        pltpu.VMEM_SHARED((B + 8, 128), f32),
        pltpu.VMEM((BW,), i32),
        pltpu.VMEM((STRIPE,), i32),
        pltpu.VMEM((CB,), i32),
        pltpu.VMEM((BK,), i32),
        pltpu.VMEM((BK,), i32),
        pltpu.VMEM((BK,), i32),
        pltpu.VMEM((BK, 128), f32),
        pltpu.VMEM((BK,), f32),
        pltpu.VMEM((16, 128), f32),
        pltpu.SemaphoreType.DMA,
        pltpu.SemaphoreType.DMA,
    ]
    if use_rel:
        scratch += [
            pltpu.VMEM_SHARED((1024, 128), f32),
            pltpu.VMEM((BK,), i32),
            pltpu.VMEM((BK, 128), f32),
        ]

    @functools.partial(
        pl.kernel,
        out_type=jax.ShapeDtypeStruct((2, B + 8, 128), f32),
        mesh=_mesh(),
        compiler_params=pltpu.CompilerParams(needs_layout_passes=False),
        scratch_types=scratch,
    )
    def pagg_kernel(*args):
        if use_rel:
            (h_hbm, src_hbm, dst_hbm, rel_hbm, invout_hbm, relT_hbm,
             bmap_hbm, slot_hbm, out_hbm,
             agg_sh, bmapv, dstrip, cpos, srcb, dstb, slotb, hrows, scal_v,
             zeros_v, semi, semh, relT_sh, relb, rrows) = args
        else:
            (h_hbm, src_hbm, dst_hbm, invout_hbm,
             bmap_hbm, slot_hbm, out_hbm,
             agg_sh, bmapv, dstrip, cpos, srcb, dstb, slotb, hrows, scal_v,
             zeros_v, semi, semh) = args

        c = lax.axis_index("c")
        s = lax.axis_index("s")
        wid = s * NC + c

        def zb(r, _):
            for j in range(8):
                zeros_v[r, pl.ds(j * LN, LN)] = jnp.zeros((LN,), f32)
            return 0

        lax.fori_loop(0, 16, zb, 0)

        def azb(k, _):
            pltpu.sync_copy(zeros_v, agg_sh.at[pl.ds(s * EROWS + k * 16, 16)])
            return 0

        lax.fori_loop(0, EROWS // 16, azb, 0)

        if use_rel:
            pltpu.sync_copy(relT_hbm.at[pl.ds(s * 64, 64)],
                            relT_sh.at[pl.ds(s * 64, 64)])
        pltpu.sync_copy(bmap_hbm, bmapv)
        base = wid * STRIPE
        pltpu.sync_copy(dst_hbm.at[pl.ds(base, STRIPE)], dstrip)
        plsc.subcore_barrier()

        def cb(v, cnt):
            off = v * LN
            d = dstrip[pl.ds(off, LN)]
            wi = jnp.minimum(lax.shift_right_logical(d, 5), BW - 1)
            w = plsc.load_gather(bmapv, [wi])
            bit = lax.shift_right_logical(w, d & 31) & 1
            m = (bit == 1) & (d < N)
            pos = jnp.full((LN,), base + off, i32) + lax.iota(i32, LN)
            plsc.store_compressed(cpos.at[pl.ds(cnt, LN)], pos, mask=m)
            return cnt + jnp.sum(jnp.where(m, 1, 0))

        cnt = lax.fori_loop(0, STRIPE // LN, cb, 0)
        for k in range(BK // LN):
            cpos[pl.ds(cnt + k * LN, LN)] = jnp.full((LN,), ER - 1, i32)
        kb = (cnt + BK - 1) // BK

        def blk(k, _):
            boff = k * BK
            pltpu.sync_copy(src_hbm.at[cpos.at[pl.ds(boff, BK)]], srcb)
            pltpu.sync_copy(dst_hbm.at[cpos.at[pl.ds(boff, BK)]], dstb)
            if use_rel:
                pltpu.sync_copy(rel_hbm.at[cpos.at[pl.ds(boff, BK)]], relb)
            pltpu.sync_copy(h_hbm.at[srcb], hrows)
            pltpu.sync_copy(invout_hbm.at[srcb], scal_v)
            pltpu.sync_copy(slot_hbm.at[dstb], slotb)
            if use_rel:
                pltpu.sync_copy(relT_sh.at[relb], rrows)
            for v in range(BK // LN):
                lane = (jnp.full((LN,), boff + v * LN, i32)
                        + lax.iota(i32, LN))
                sl = slotb[pl.ds(v * LN, LN)]
                slotb[pl.ds(v * LN, LN)] = jnp.where(lane < cnt, sl, B)

            def rowm(gr, _):
                scv = scal_v[pl.ds(gr * LN, LN)]
                for r16 in range(LN):
                    r = gr * LN + r16
                    sc = jnp.broadcast_to(scv[r16], (LN,))
                    for j in range(8):
                        x = hrows[r, pl.ds(j * LN, LN)] * sc
                        if use_rel:
                            x = x * rrows[r, pl.ds(j * LN, LN)]
                        hrows[r, pl.ds(j * LN, LN)] = x
                return 0

            lax.fori_loop(0, BK // LN, rowm, 0)
            pltpu.sync_copy(hrows, agg_sh.at[slotb], add=True)
            return 0

        lax.fori_loop(0, kb, blk, 0)
        plsc.subcore_barrier()

        def wo(k, _):
            r0 = s * EROWS + k * BK
            pltpu.sync_copy(agg_sh.at[pl.ds(r0, BK)], hrows)
            pltpu.sync_copy(hrows, out_hbm.at[c, pl.ds(r0, BK)])
            return 0

        lax.fori_loop(0, EROWS // BK, wo, 0)

    return pagg_kernel


def _make_final_kernel():
    B = 4096
    PW = B // (NC * NS)

    @functools.partial(
        pl.kernel,
        out_type=(
            jax.ShapeDtypeStruct((B, 128), f32),
            jax.ShapeDtypeStruct((B, 128), f32),
        ),
        mesh=_mesh(),
        compiler_params=pltpu.CompilerParams(needs_layout_passes=False),
        scratch_types=[
            pltpu.VMEM((PW,), i32),
            pltpu.VMEM((PW,), i32),
            pltpu.VMEM((PW,), f32),
            pltpu.VMEM((PW, 128), f32),
            pltpu.VMEM((PW, 128), f32),
            pltpu.VMEM((PW, 128), f32),
            pltpu.VMEM((PW, 128), f32),
        ],
    )
    def final_kernel(ht_hbm, hg_hbm, at0_hbm, at1_hbm, ag0_hbm, ag1_hbm,
                     it_hbm, ig_hbm, st_hbm, sg_hbm,
                     er_hbm, ee_hbm, seed_hbm, rb_hbm,
                     n_hbm, e_hbm, ia_v, ib_v, sc_v, a_v, b_v, h_v, n_v):
        c = lax.axis_index("c")
        s = lax.axis_index("s")
        wid = s * NC + c
        base = wid * PW

        pltpu.sync_copy(seed_hbm.at[pl.ds(base, PW)], ia_v)

        def half_path(agg0, agg1, inv_hbm, slot_hbm, hop1_hbm, acc):
            pltpu.sync_copy(inv_hbm.at[ia_v], sc_v)
            pltpu.sync_copy(slot_hbm.at[ia_v], ib_v)
            pltpu.sync_copy(agg0.at[ib_v], a_v)
            pltpu.sync_copy(agg1.at[ib_v], b_v)
            pltpu.sync_copy(hop1_hbm.at[ia_v], h_v)

            def rowm(gr, _):
                scv = sc_v[pl.ds(gr * LN, LN)]
                for r16 in range(LN):
                    r = gr * LN + r16
                    iv = jnp.broadcast_to(scv[r16], (LN,))
                    for j in range(8):
                        x = ((a_v[r, pl.ds(j * LN, LN)]
                              + b_v[r, pl.ds(j * LN, LN)]) * iv
                             + h_v[r, pl.ds(j * LN, LN)]) * 0.5
                        if acc:
                            x = x + n_v[r, pl.ds(j * LN, LN)]
                        n_v[r, pl.ds(j * LN, LN)] = x
                return 0

            lax.fori_loop(0, PW // LN, rowm, 0)

        half_path(at0_hbm, at1_hbm, it_hbm, st_hbm, ht_hbm, False)

        def ob(idx, _):
            sv = ia_v[pl.ds(idx * LN, LN)]
            ia_v[pl.ds(idx * LN, LN)] = lax.shift_right_logical(sv, 2)
            return 0

        lax.fori_loop(0, PW // LN, ob, 0)
        half_path(ag0_hbm, ag1_hbm, ig_hbm, sg_hbm, hg_hbm, True)
        pltpu.sync_copy(n_v, n_hbm.at[pl.ds(base, PW)])

        pltpu.sync_copy(rb_hbm.at[pl.ds(base, PW)], ia_v)
        pltpu.sync_copy(er_hbm.at[ia_v], a_v)
        pltpu.sync_copy(ee_hbm.at[ia_v], b_v)

        def rowe(r, _):
            for j in range(8):
                a_v[r, pl.ds(j * LN, LN)] = (
                    a_v[r, pl.ds(j * LN, LN)] * 0.5
                    + b_v[r, pl.ds(j * LN, LN)] * 0.5)
            return 0

        lax.fori_loop(0, PW, rowe, 0)
        pltpu.sync_copy(a_v, e_hbm.at[pl.ds(base, PW)])

    return final_kernel


_NT, _NG, _NRL = 200000, 50000, 1000
_C = 8192
_NR_T, _NCH_T = 204800, 25
_NR_G, _NCH_G = 57344, 7
_NR_R, _NCH_R = 1024, 1
_ER_BIG, _ER_REL = 409600, 16384
_BWT, _BWG = 6400, 2048
_SLT, _SLG = 206848, 59392

_deg_t = _make_deg_kernel(_NR_T, _NR_T + 2048, _ER_BIG)
_deg_g = _make_deg_kernel(_NR_G, _NR_G + 2048, _ER_BIG)
_deg_r = _make_deg_kernel(_NR_R, _NR_R + 2048, _ER_REL)
_gcn_t = _make_gcn_kernel(_NT, _NR_T, _C, _NCH_T, _ER_BIG, True, True)
_gcn_g = _make_gcn_kernel(_NG, _NR_G, _C, _NCH_G, _ER_BIG, True, True)
_gcn_r = _make_gcn_kernel(_NR_R, _NR_R, _NR_R, _NCH_R, _ER_REL, False, False)
_prep = _make_seed_prep_kernel(_BWT, _BWG, _SLT, _SLG)
_pagg_t = _make_pruned_agg_kernel(_NT, _BWT, _ER_BIG, True)
_pagg_g = _make_pruned_agg_kernel(_NG, _BWG, _ER_BIG, True)
_final = _make_final_kernel()


def _pad_edges(src, dst, rel, er, dummy_src, dummy_dst):
    pe = er - src.shape[0]
    out = (jnp.concatenate([src, jnp.full((pe,), dummy_src, i32)]),
           jnp.concatenate([dst, jnp.full((pe,), dummy_dst, i32)]))
    if rel is not None:
        out += (jnp.concatenate([rel, jnp.zeros((pe,), i32)]),)
    return out


def kernel(g_edge_index, g_edge_rel, glob_edge_index, glob_edge_rel,
           rel_edge_index, seed_nodes, relation_batch, neighbor_batch_size,
           node_emb, global_emb, edge_emb):
    del neighbor_batch_size
    tsrc, tdst, trel = _pad_edges(
        g_edge_index[0], g_edge_index[1], g_edge_rel, _ER_BIG, _NT, _NR_T)
    gsrc, gdst, grel = _pad_edges(
        glob_edge_index[0], glob_edge_index[1], glob_edge_rel, _ER_BIG,
        _NG, _NR_G)
    rsrc, rdst = _pad_edges(
        rel_edge_index[0], rel_edge_index[1], None, _ER_REL, _NRL, _NRL)
    ee_pad = jnp.concatenate(
        [edge_emb, jnp.zeros((_NR_R - _NRL, 128), f32)], axis=0)

    t_inv = _deg_t(jnp.stack([tsrc, tdst]))
    g_inv = _deg_g(jnp.stack([gsrc, gdst]))
    r_inv = _deg_r(jnp.stack([rsrc, rdst]))
    t_io, t_ii = t_inv[0], t_inv[1]
    g_io, g_ii = g_inv[0], g_inv[1]
    r_io, r_ii = r_inv[0], r_inv[1]

    bmap_t, bmap_g, slot_t, slot_g = _prep(seed_nodes)

    ht1 = _gcn_t(node_emb, tsrc, tdst, trel, t_io, t_ii, ee_pad)
    hg1 = _gcn_g(global_emb, gsrc, gdst, grel, g_io, g_ii, ee_pad)
    at = _pagg_t(ht1, tsrc, tdst, trel, t_io, ee_pad, bmap_t, slot_t)
    ag = _pagg_g(hg1, gsrc, gdst, grel, g_io, ee_pad, bmap_g, slot_g)
    er = _gcn_r(ee_pad, rsrc, rdst, r_io, r_ii)

    n, e = _final(ht1, hg1, at[0], at[1], ag[0], ag[1], t_ii, g_ii,
                  slot_t, slot_g, er, ee_pad, seed_nodes, relation_batch)
    return n, e
